# baseline (speedup 1.0000x reference)
import functools

import jax
import jax.numpy as jnp
from jax import lax
from jax.experimental import pallas as pl
from jax.experimental.pallas import tpu as pltpu
from jax.experimental.pallas import tpu_sc as plsc

DIM = 64
CAP = 500000
NQ = 32
KK = 8
BLK = 8192
GRID = (CAP + BLK - 1) // BLK

_NEG_INF = float("-inf")
_BIG_F = float(2**25)


def _topk_body(q_ref, kt_ref, scores_out, idx_out, rv_ref, ri_ref, qn_ref):
    t = pl.program_id(0)

    @pl.when(t == 0)
    def _init():
        rv_ref[...] = jnp.full((NQ, KK), _NEG_INF, jnp.float32)
        ri_ref[...] = jnp.full((NQ, KK), _BIG_F, jnp.float32)
        q = q_ref[...]
        qn_ref[...] = q / jnp.maximum(
            jnp.sqrt(jnp.sum(q * q, axis=1, keepdims=True)), 1e-12)

    qn = qn_ref[...]
    kt = kt_ref[...]
    ss = jnp.sum(kt * kt, axis=0, keepdims=True)
    kn = kt / jnp.maximum(jnp.sqrt(ss), 1e-12)
    simn = lax.dot_general(
        qn, kn, (((1,), (0,)), ((), ())),
        preferred_element_type=jnp.float32)

    col = (lax.broadcasted_iota(jnp.int32, (NQ, BLK), 1).astype(jnp.float32)
           + jnp.float32(t * BLK))
    simn = jnp.where(col < float(CAP), simn, _NEG_INF)

    comb_v = jnp.concatenate([rv_ref[...], simn], axis=1)
    comb_i = jnp.concatenate([ri_ref[...], col], axis=1)

    vals, idxs = [], []
    for _ in range(KK):
        m = jnp.max(comb_v, axis=1)
        eq = comb_v == m[:, None]
        ci = jnp.min(jnp.where(eq, comb_i, _BIG_F), axis=1)
        vals.append(m)
        idxs.append(ci)
        comb_v = jnp.where(comb_i == ci[:, None], _NEG_INF, comb_v)
    rv = jnp.stack(vals, axis=1)
    ri = jnp.stack(idxs, axis=1)
    rv_ref[...] = rv
    ri_ref[...] = ri

    @pl.when(t == GRID - 1)
    def _fin():
        scores_out[...] = rv
        idx_out[...] = ri.astype(jnp.int32)


_topk_call = pl.pallas_call(
    _topk_body,
    grid=(GRID,),
    in_specs=[
        pl.BlockSpec((NQ, DIM), lambda t: (0, 0)),
        pl.BlockSpec((DIM, BLK), lambda t: (0, t)),
    ],
    out_specs=[
        pl.BlockSpec((NQ, KK), lambda t: (0, 0)),
        pl.BlockSpec((NQ, KK), lambda t: (0, 0)),
    ],
    out_shape=[
        jax.ShapeDtypeStruct((NQ, KK), jnp.float32),
        jax.ShapeDtypeStruct((NQ, KK), jnp.int32),
    ],
    scratch_shapes=[
        pltpu.VMEM((NQ, KK), jnp.float32),
        pltpu.VMEM((NQ, KK), jnp.float32),
        pltpu.VMEM((NQ, DIM), jnp.float32),
    ],
    compiler_params=pltpu.CompilerParams(
        dimension_semantics=("arbitrary",)),
)


def _kgather_body(jb_ref, jl_ref, kt_ref, out_ref):
    i = pl.program_id(0)

    @pl.when(i % 128 == 0)
    def _zero():
        out_ref[...] = jnp.zeros((DIM, 128), jnp.float32)

    blk = kt_ref[...]
    jl = jl_ref[i]
    lane = lax.broadcasted_iota(jnp.int32, (1, 128), 1)
    colsel = jnp.where(lane == jl, blk, 0.0)
    ss = jnp.sum(colsel * colsel)
    rn = 1.0 / jnp.maximum(jnp.sqrt(ss), 1e-12)
    cvec = jnp.sum(colsel, axis=1, keepdims=True) * rn
    out_ref[...] = out_ref[...] + jnp.where(lane == (i % 128), cvec, 0.0)


_kgather_call = pl.pallas_call(
    _kgather_body,
    grid_spec=pltpu.PrefetchScalarGridSpec(
        num_scalar_prefetch=2,
        grid=(NQ * KK,),
        in_specs=[
            pl.BlockSpec((DIM, 128), lambda i, jb, jl: (0, jb[i])),
        ],
        out_specs=pl.BlockSpec((DIM, 128), lambda i, jb, jl: (0, i // 128)),
    ),
    out_shape=jax.ShapeDtypeStruct((DIM, NQ * KK), jnp.float32),
    compiler_params=pltpu.CompilerParams(
        dimension_semantics=("arbitrary",)),
)


_NC, _NS = 2, 16
_NW = _NC * _NS
ROWS = NQ * KK
RPW = ROWS // _NW


@functools.cache
def _make_sc_gather():
    mesh = plsc.VectorSubcoreMesh(core_axis_name="c", subcore_axis_name="s")

    @functools.partial(
        pl.kernel,
        mesh=mesh,
        out_type=jax.ShapeDtypeStruct((ROWS, DIM), jnp.float32),
        scratch_types=[
            pltpu.VMEM((16,), jnp.int32),
            pltpu.VMEM((16,), jnp.int32),
            pltpu.VMEM((16, 2 * DIM), jnp.float32),
            pltpu.VMEM((RPW, DIM), jnp.float32),
            pltpu.SemaphoreType.DMA,
        ],
        compiler_params=pltpu.CompilerParams(use_tc_tiling_on_sc=True),
    )
    def _sc_gather(v2_hbm, idx_hbm, outv_hbm, idx16, slb, vsl, vrows, sem):
        wid = lax.axis_index("s") * _NC + lax.axis_index("c")
        base = wid * RPW
        pltpu.sync_copy(idx_hbm.at[pl.ds(base, RPW)], idx16.at[pl.ds(0, RPW)])
        iv = jnp.minimum(jnp.maximum(idx16[...], 0), CAP - 1)
        par = iv & 1
        slb[...] = iv >> 1

        pltpu.async_copy(v2_hbm.at[slb], vsl, sem).wait()

        dn_b = lax.GatherDimensionNumbers(
            offset_dims=(), collapsed_slice_dims=(0,), start_index_map=(0,))
        for r in range(RPW):
            pv = lax.gather(
                par, jnp.full((16, 1), r, jnp.int32), dn_b, slice_sizes=(1,),
                mode=lax.GatherScatterMode.PROMISE_IN_BOUNDS)
            e = pv.astype(jnp.float32)
            ne = 1.0 - e
            for c in range(DIM // 16):
                lo = vsl[r, pl.ds(c * 16, 16)]
                hi = vsl[r, pl.ds(DIM + c * 16, 16)]
                vrows[r, pl.ds(c * 16, 16)] = lo * ne + hi * e

        pltpu.sync_copy(vrows, outv_hbm.at[pl.ds(base, RPW)])

    return _sc_gather


@functools.cache
def _make_sc_prewarm():
    mesh = plsc.VectorSubcoreMesh(core_axis_name="c", subcore_axis_name="s")

    @functools.partial(
        pl.kernel,
        mesh=mesh,
        out_type=jax.ShapeDtypeStruct((16,), jnp.float32),
        scratch_types=[
            pltpu.VMEM((1, 2 * DIM), jnp.float32),
            pltpu.VMEM((16,), jnp.float32),
            pltpu.SemaphoreType.DMA,
        ],
        compiler_params=pltpu.CompilerParams(use_tc_tiling_on_sc=True),
    )
    def _sc_prewarm(v_hbm, out_hbm, vr, acc, sem):
        wid = lax.axis_index("s") * _NC + lax.axis_index("c")

        @pl.when(wid == 0)
        def _():
            pltpu.async_copy(v_hbm.at[pl.ds(0, 1)], vr, sem).wait()
            acc[...] = vr[0, pl.ds(0, 16)]
            pltpu.sync_copy(acc, out_hbm)

    return _sc_prewarm


def kernel(k, v, query, top_k):
    del top_k
    kt = jnp.swapaxes(k, 0, 1)
    v2 = v.reshape(CAP // 2, 2 * DIM)
    warm = _make_sc_prewarm()(v2)
    scores, idx = _topk_call(query, kt)
    idx = lax.optimization_barrier((idx, warm))[0]
    idxf = idx.reshape(-1)
    outv = _make_sc_gather()(v2, idxf)
    outk_t = _kgather_call(idxf // 128, idxf % 128, kt)
    outk = jnp.swapaxes(outk_t, 0, 1)
    return (outk.reshape(NQ, KK, DIM),
            outv.reshape(NQ, KK, DIM),
            scores)


# speedup vs baseline: 1.0353x; 1.0353x over previous
import functools

import jax
import jax.numpy as jnp
from jax import lax
from jax.experimental import pallas as pl
from jax.experimental.pallas import tpu as pltpu
from jax.experimental.pallas import tpu_sc as plsc

DIM = 64
CAP = 500000
NQ = 32
KK = 8
BLK = 16384
GRID = (CAP + BLK - 1) // BLK

_NEG_INF = float("-inf")
_BIG_F = float(2**25)


def _topk_body(q_ref, kt_ref, scores_out, idx_out, rv_ref, ri_ref, qn_ref):
    t = pl.program_id(0)

    @pl.when(t == 0)
    def _init():
        rv_ref[...] = jnp.full((NQ, KK), _NEG_INF, jnp.float32)
        ri_ref[...] = jnp.full((NQ, KK), _BIG_F, jnp.float32)
        q = q_ref[...]
        qn_ref[...] = q / jnp.maximum(
            jnp.sqrt(jnp.sum(q * q, axis=1, keepdims=True)), 1e-12)

    qn = qn_ref[...]
    kt = kt_ref[...]
    ss = jnp.sum(kt * kt, axis=0, keepdims=True)
    kn = kt / jnp.maximum(jnp.sqrt(ss), 1e-12)
    simn = lax.dot_general(
        qn, kn, (((1,), (0,)), ((), ())),
        preferred_element_type=jnp.float32)

    col = (lax.broadcasted_iota(jnp.int32, (NQ, BLK), 1).astype(jnp.float32)
           + jnp.float32(t * BLK))
    simn = jnp.where(col < float(CAP), simn, _NEG_INF)

    comb_v = jnp.concatenate([rv_ref[...], simn], axis=1)
    comb_i = jnp.concatenate([ri_ref[...], col], axis=1)

    vals, idxs = [], []
    for _ in range(KK):
        m = jnp.max(comb_v, axis=1)
        eq = comb_v == m[:, None]
        ci = jnp.min(jnp.where(eq, comb_i, _BIG_F), axis=1)
        vals.append(m)
        idxs.append(ci)
        comb_v = jnp.where(comb_i == ci[:, None], _NEG_INF, comb_v)
    rv = jnp.stack(vals, axis=1)
    ri = jnp.stack(idxs, axis=1)
    rv_ref[...] = rv
    ri_ref[...] = ri

    @pl.when(t == GRID - 1)
    def _fin():
        scores_out[...] = rv
        idx_out[...] = ri.astype(jnp.int32)


_topk_call = pl.pallas_call(
    _topk_body,
    grid=(GRID,),
    in_specs=[
        pl.BlockSpec((NQ, DIM), lambda t: (0, 0)),
        pl.BlockSpec((DIM, BLK), lambda t: (0, t)),
    ],
    out_specs=[
        pl.BlockSpec((NQ, KK), lambda t: (0, 0)),
        pl.BlockSpec((NQ, KK), lambda t: (0, 0)),
    ],
    out_shape=[
        jax.ShapeDtypeStruct((NQ, KK), jnp.float32),
        jax.ShapeDtypeStruct((NQ, KK), jnp.int32),
    ],
    scratch_shapes=[
        pltpu.VMEM((NQ, KK), jnp.float32),
        pltpu.VMEM((NQ, KK), jnp.float32),
        pltpu.VMEM((NQ, DIM), jnp.float32),
    ],
    compiler_params=pltpu.CompilerParams(
        dimension_semantics=("arbitrary",)),
)


def _kgather_body(jb_ref, jl_ref, kt_ref, out_ref):
    i = pl.program_id(0)

    @pl.when(i % 128 == 0)
    def _zero():
        out_ref[...] = jnp.zeros((DIM, 128), jnp.float32)

    blk = kt_ref[...]
    jl = jl_ref[i]
    lane = lax.broadcasted_iota(jnp.int32, (1, 128), 1)
    colsel = jnp.where(lane == jl, blk, 0.0)
    ss = jnp.sum(colsel * colsel)
    rn = 1.0 / jnp.maximum(jnp.sqrt(ss), 1e-12)
    cvec = jnp.sum(colsel, axis=1, keepdims=True) * rn
    out_ref[...] = out_ref[...] + jnp.where(lane == (i % 128), cvec, 0.0)


_kgather_call = pl.pallas_call(
    _kgather_body,
    grid_spec=pltpu.PrefetchScalarGridSpec(
        num_scalar_prefetch=2,
        grid=(NQ * KK,),
        in_specs=[
            pl.BlockSpec((DIM, 128), lambda i, jb, jl: (0, jb[i])),
        ],
        out_specs=pl.BlockSpec((DIM, 128), lambda i, jb, jl: (0, i // 128)),
    ),
    out_shape=jax.ShapeDtypeStruct((DIM, NQ * KK), jnp.float32),
    compiler_params=pltpu.CompilerParams(
        dimension_semantics=("arbitrary",)),
)


_NC, _NS = 2, 16
_NW = _NC * _NS
ROWS = NQ * KK
RPW = ROWS // _NW


@functools.cache
def _make_sc_gather():
    mesh = plsc.VectorSubcoreMesh(core_axis_name="c", subcore_axis_name="s")

    @functools.partial(
        pl.kernel,
        mesh=mesh,
        out_type=jax.ShapeDtypeStruct((ROWS, DIM), jnp.float32),
        scratch_types=[
            pltpu.VMEM((16,), jnp.int32),
            pltpu.VMEM((16,), jnp.int32),
            pltpu.VMEM((16, 2 * DIM), jnp.float32),
            pltpu.VMEM((RPW, DIM), jnp.float32),
            pltpu.SemaphoreType.DMA,
        ],
        compiler_params=pltpu.CompilerParams(use_tc_tiling_on_sc=True),
    )
    def _sc_gather(v2_hbm, idx_hbm, outv_hbm, idx16, slb, vsl, vrows, sem):
        wid = lax.axis_index("s") * _NC + lax.axis_index("c")
        base = wid * RPW
        pltpu.sync_copy(idx_hbm.at[pl.ds(base, RPW)], idx16.at[pl.ds(0, RPW)])
        iv = jnp.minimum(jnp.maximum(idx16[...], 0), CAP - 1)
        par = iv & 1
        slb[...] = iv >> 1

        pltpu.async_copy(v2_hbm.at[slb], vsl, sem).wait()

        dn_b = lax.GatherDimensionNumbers(
            offset_dims=(), collapsed_slice_dims=(0,), start_index_map=(0,))
        for r in range(RPW):
            pv = lax.gather(
                par, jnp.full((16, 1), r, jnp.int32), dn_b, slice_sizes=(1,),
                mode=lax.GatherScatterMode.PROMISE_IN_BOUNDS)
            e = pv.astype(jnp.float32)
            ne = 1.0 - e
            for c in range(DIM // 16):
                lo = vsl[r, pl.ds(c * 16, 16)]
                hi = vsl[r, pl.ds(DIM + c * 16, 16)]
                vrows[r, pl.ds(c * 16, 16)] = lo * ne + hi * e

        pltpu.sync_copy(vrows, outv_hbm.at[pl.ds(base, RPW)])

    return _sc_gather


@functools.cache
def _make_sc_prewarm():
    mesh = plsc.VectorSubcoreMesh(core_axis_name="c", subcore_axis_name="s")

    @functools.partial(
        pl.kernel,
        mesh=mesh,
        out_type=jax.ShapeDtypeStruct((16,), jnp.float32),
        scratch_types=[
            pltpu.VMEM((1, 2 * DIM), jnp.float32),
            pltpu.VMEM((16,), jnp.float32),
            pltpu.SemaphoreType.DMA,
        ],
        compiler_params=pltpu.CompilerParams(use_tc_tiling_on_sc=True),
    )
    def _sc_prewarm(v_hbm, out_hbm, vr, acc, sem):
        wid = lax.axis_index("s") * _NC + lax.axis_index("c")

        @pl.when(wid == 0)
        def _():
            pltpu.async_copy(v_hbm.at[pl.ds(0, 1)], vr, sem).wait()
            acc[...] = vr[0, pl.ds(0, 16)]
            pltpu.sync_copy(acc, out_hbm)

    return _sc_prewarm


def kernel(k, v, query, top_k):
    del top_k
    kt = jnp.swapaxes(k, 0, 1)
    v2 = v.reshape(CAP // 2, 2 * DIM)
    warm = _make_sc_prewarm()(v2)
    scores, idx = _topk_call(query, kt)
    idx = lax.optimization_barrier((idx, warm))[0]
    idxf = idx.reshape(-1)
    outv = _make_sc_gather()(v2, idxf)
    outk_t = _kgather_call(idxf // 128, idxf % 128, kt)
    outk = jnp.swapaxes(outk_t, 0, 1)
    return (outk.reshape(NQ, KK, DIM),
            outv.reshape(NQ, KK, DIM),
            scores)
